# Initial kernel scaffold; baseline (speedup 1.0000x reference)
import jax
import jax.numpy as jnp
from jax.experimental import pallas as pl


def kernel(vertex_features, edge_index, pairs_cells, pairs_cols, pairs_rows, targets_cells, targets_cols, targets_rows, W_gcnn, b_gcnn, W_h1, b_h1, W_h2, b_h2):
    raise NotImplementedError("write your pallas kernel here")


# speedup vs baseline: 1.9324x; 1.9324x over previous
import functools

import jax
import jax.numpy as jnp
from jax import lax
from jax.experimental import pallas as pl
from jax.experimental.pallas import tpu as pltpu
from jax.experimental.pallas import tpu_sc as plsc

N = 10000
NPAD = 10240
E = 320000
P = 100000
DP = 128
DH = DP // 2
PPAD = 102400
PTOT = 3 * PPAD
NC, NS = 2, 16
NW = NC * NS
EPS = E // NS
RPT = NPAD // NS
PPW = PTOT // NW
KE = 80
KP = 80


def _linear(vf, Wp, bp):
    BM = 400

    def body(x_ref, w_ref, b_ref, o_ref):
        res = (
            jnp.dot(x_ref[...], w_ref[...], preferred_element_type=jnp.float32)
            + b_ref[...]
        )
        o_ref[0] = res[:, :DH]
        o_ref[1] = res[:, DH:]

    return pl.pallas_call(
        body,
        grid=(N // BM,),
        in_specs=[
            pl.BlockSpec((BM, DP), lambda i: (i, 0)),
            pl.BlockSpec((DP, DP), lambda i: (0, 0)),
            pl.BlockSpec((1, DP), lambda i: (0, 0)),
        ],
        out_specs=pl.BlockSpec((NC, BM, DH), lambda i: (0, i, 0)),
        out_shape=jax.ShapeDtypeStruct((NC, N, DH), jnp.float32),
    )(vf, Wp, bp)


def _edge_agg(h2, src, dst):
    mesh = plsc.VectorSubcoreMesh(core_axis_name="c", subcore_axis_name="s")
    nchunk = EPS // KE

    @functools.partial(
        pl.kernel,
        out_type=jax.ShapeDtypeStruct((NC, NPAD, DH), jnp.float32),
        mesh=mesh,
        compiler_params=pltpu.CompilerParams(use_tc_tiling_on_sc=False),
        scratch_types=[
            pltpu.VMEM((KE,), jnp.int32),
            pltpu.VMEM((KE,), jnp.int32),
            pltpu.VMEM((RPT, DH), jnp.float32),
            pltpu.VMEM_SHARED((NPAD, DH), jnp.float32),
            pltpu.SemaphoreType.DMA,
        ],
    )
    def k(h_hbm, src_hbm, dst_hbm, out_hbm, sidx, didx, buf, acc, sem):
        cid = lax.axis_index("c")
        sid = lax.axis_index("s")

        @pl.loop(0, RPT)
        def _(r):
            for l in range(DH // 16):
                buf[r, pl.ds(l * 16, 16)] = jnp.zeros((16,), jnp.float32)

        pltpu.sync_copy(buf, acc.at[pl.ds(sid * RPT, RPT)])
        plsc.subcore_barrier()

        @pl.loop(0, nchunk)
        def _(it):
            base = sid * EPS + it * KE
            pltpu.sync_copy(src_hbm.at[pl.ds(base, KE)], sidx)
            pltpu.sync_copy(dst_hbm.at[pl.ds(base, KE)], didx)
            pltpu.async_copy(
                h_hbm.at[cid].at[sidx], buf.at[pl.ds(0, KE)], sem
            ).wait()
            pltpu.sync_copy(buf.at[pl.ds(0, KE)], acc.at[didx], add=True)

        plsc.subcore_barrier()
        pltpu.sync_copy(acc.at[pl.ds(sid * RPT, RPT)], buf)
        pltpu.sync_copy(buf, out_hbm.at[cid, pl.ds(sid * RPT, RPT)])

    return k(h2, src, dst)


def _finalize_gf(partials):
    BM = 512

    def body(p_ref, o_ref):
        x0 = p_ref[0]
        x1 = p_ref[1]
        li = lax.broadcasted_iota(jnp.int32, (BM, DH), 1)
        deg = jnp.sum(jnp.where(li == 100 - DH, x1, 0.0), axis=1, keepdims=True)
        r = 1.0 / jnp.maximum(deg, 1.0)
        o_ref[...] = jnp.concatenate(
            [jnp.maximum(x0 * r, 0.0), jnp.maximum(x1 * r, 0.0)], axis=1
        )

    return pl.pallas_call(
        body,
        grid=(NPAD // BM,),
        in_specs=[pl.BlockSpec((NC, BM, DH), lambda i: (0, i, 0))],
        out_specs=pl.BlockSpec((BM, DP), lambda i: (i, 0)),
        out_shape=jax.ShapeDtypeStruct((NPAD, DP), jnp.float32),
    )(partials)


def _pair_diff(gf, pa, pb):
    mesh = plsc.VectorSubcoreMesh(core_axis_name="c", subcore_axis_name="s")
    nchunk = PPW // KP

    @functools.partial(
        pl.kernel,
        out_type=jax.ShapeDtypeStruct((PTOT, DP), jnp.float32),
        mesh=mesh,
        scratch_types=[
            pltpu.VMEM((KP,), jnp.int32),
            pltpu.VMEM((KP,), jnp.int32),
            pltpu.VMEM((KP, DP), jnp.float32),
            pltpu.VMEM((KP, DP), jnp.float32),
            pltpu.SemaphoreType.DMA,
            pltpu.SemaphoreType.DMA,
        ],
    )
    def k(gf_hbm, pa_hbm, pb_hbm, out_hbm, ia, ib, bufa, bufb, sema, semb):
        cid = lax.axis_index("c")
        sid = lax.axis_index("s")
        w = cid * NS + sid

        @pl.loop(0, nchunk)
        def _(it):
            base = w * PPW + it * KP
            pltpu.sync_copy(pa_hbm.at[pl.ds(base, KP)], ia)
            pltpu.sync_copy(pb_hbm.at[pl.ds(base, KP)], ib)
            ca = pltpu.async_copy(gf_hbm.at[ia], bufa, sema)
            cb = pltpu.async_copy(gf_hbm.at[ib], bufb, semb)
            ca.wait()
            cb.wait()

            @pl.loop(0, KP)
            def _(r):
                for l in range(DP // 16):
                    sl = pl.ds(l * 16, 16)
                    bufa[r, sl] = jnp.abs(bufa[r, sl] - bufb[r, sl])

            pltpu.sync_copy(bufa, out_hbm.at[pl.ds(base, KP)])

    return k(gf, pa, pb)


def _head_loss(dmat, W1p, b1p, w2p, b2, tvec):
    BM = 1024
    G = PTOT // BM
    SCALE = 1.0 / P

    def body(d_ref, w1_ref, b1_ref, w2_ref, b2_ref, t_ref, o_ref):
        i = pl.program_id(0)
        d = d_ref[...]
        hdn = jnp.maximum(
            jnp.dot(d, w1_ref[...], preferred_element_type=jnp.float32)
            + b1_ref[...],
            0.0,
        )
        lg = jnp.sum(hdn * w2_ref[...], axis=1, keepdims=True) + b2_ref[...]
        t = t_ref[...]
        row = i * BM + lax.broadcasted_iota(jnp.int32, (BM, 1), 0)
        s = (row >= PPAD).astype(jnp.int32) + (row >= 2 * PPAD).astype(jnp.int32)
        local = row - s * PPAD
        wgt = jnp.where(local < P, SCALE, 0.0)
        bce = jnp.maximum(lg, 0.0) - lg * t + jnp.log1p(jnp.exp(-jnp.abs(lg)))
        part = jnp.reshape(jnp.sum(bce * wgt), (1, 1))

        @pl.when(i == 0)
        def _():
            o_ref[...] = part

        @pl.when(i > 0)
        def _():
            o_ref[...] += part

    return pl.pallas_call(
        body,
        grid=(G,),
        in_specs=[
            pl.BlockSpec((BM, DP), lambda i: (i, 0)),
            pl.BlockSpec((DP, DP), lambda i: (0, 0)),
            pl.BlockSpec((1, DP), lambda i: (0, 0)),
            pl.BlockSpec((1, DP), lambda i: (0, 0)),
            pl.BlockSpec((1, 1), lambda i: (0, 0)),
            pl.BlockSpec((BM, 1), lambda i: (i, 0)),
        ],
        out_specs=pl.BlockSpec((1, 1), lambda i: (0, 0)),
        out_shape=jax.ShapeDtypeStruct((1, 1), jnp.float32),
    )(dmat, W1p, b1p, w2p, b2, tvec)


def kernel(vertex_features, edge_index, pairs_cells, pairs_cols, pairs_rows,
           targets_cells, targets_cols, targets_rows,
           W_gcnn, b_gcnn, W_h1, b_h1, W_h2, b_h2):
    f32 = jnp.float32
    src = edge_index[0]
    dst = edge_index[1]

    Wp = jnp.pad(W_gcnn, ((0, 0), (0, DP - 100)))
    bp = jnp.concatenate(
        [b_gcnn, jnp.ones((1,), f32), jnp.zeros((DP - 101,), f32)]
    ).reshape(1, DP)
    W1p = jnp.pad(W_h1, ((0, DP - 100), (0, DP - 50)))
    b1p = jnp.pad(b_h1, (0, DP - 50)).reshape(1, DP)
    w2p = jnp.pad(W_h2[:, 0], (0, DP - 50)).reshape(1, DP)
    b2 = b_h2.reshape(1, 1)

    def padset(x):
        return jnp.pad(x, (0, PPAD - P))

    pa = jnp.concatenate(
        [padset(pairs_cells[:, 0]), padset(pairs_cols[:, 0]), padset(pairs_rows[:, 0])]
    )
    pb = jnp.concatenate(
        [padset(pairs_cells[:, 1]), padset(pairs_cols[:, 1]), padset(pairs_rows[:, 1])]
    )
    tvec = jnp.concatenate(
        [
            padset(targets_cells.astype(f32)),
            padset(targets_cols.astype(f32)),
            padset(targets_rows.astype(f32)),
        ]
    ).reshape(PTOT, 1)

    h2 = _linear(vertex_features, Wp, bp)
    partials = _edge_agg(h2, src, dst)
    gf = _finalize_gf(partials)
    dmat = _pair_diff(gf, pa, pb)
    out = _head_loss(dmat, W1p, b1p, w2p, b2, tvec)
    return out[0, 0]


# speedup vs baseline: 2.4072x; 1.2457x over previous
import functools

import jax
import jax.numpy as jnp
from jax import lax
from jax.experimental import pallas as pl
from jax.experimental.pallas import tpu as pltpu
from jax.experimental.pallas import tpu_sc as plsc

N = 10000
NPAD = 10240
E = 320000
P = 100000
DP = 128
DH = DP // 2
PPAD = 102400
NC, NS = 2, 16
NW = NC * NS
EPS = E // NS
RPT = NPAD // NS
PPW = PPAD // NW
KE = 80
KP = 80


def _linear(vf, Wp, bp):
    BM = 400

    def body(x_ref, w_ref, b_ref, o_ref):
        res = (
            jnp.dot(x_ref[...], w_ref[...], preferred_element_type=jnp.float32)
            + b_ref[...]
        )
        o_ref[0] = res[:, :DH]
        o_ref[1] = res[:, DH:]

    return pl.pallas_call(
        body,
        grid=(N // BM,),
        in_specs=[
            pl.BlockSpec((BM, DP), lambda i: (i, 0)),
            pl.BlockSpec((DP, DP), lambda i: (0, 0)),
            pl.BlockSpec((1, DP), lambda i: (0, 0)),
        ],
        out_specs=pl.BlockSpec((NC, BM, DH), lambda i: (0, i, 0)),
        out_shape=jax.ShapeDtypeStruct((NC, N, DH), jnp.float32),
    )(vf, Wp, bp)


def _edge_agg(h2, src, dst):
    mesh = plsc.VectorSubcoreMesh(core_axis_name="c", subcore_axis_name="s")
    nchunk = EPS // KE

    @functools.partial(
        pl.kernel,
        out_type=jax.ShapeDtypeStruct((NC, NPAD, DH), jnp.float32),
        mesh=mesh,
        compiler_params=pltpu.CompilerParams(use_tc_tiling_on_sc=False),
        scratch_types=[
            pltpu.VMEM((2, KE), jnp.int32),
            pltpu.VMEM((2, KE), jnp.int32),
            pltpu.VMEM((2, KE, DH), jnp.float32),
            pltpu.VMEM((RPT, DH), jnp.float32),
            pltpu.VMEM_SHARED((NPAD, DH), jnp.float32),
            pltpu.SemaphoreType.DMA,
            pltpu.SemaphoreType.DMA,
        ],
    )
    def k(h_hbm, src_hbm, dst_hbm, out_hbm, sidx, didx, rows, buf, acc,
          sem0, sem1):
        cid = lax.axis_index("c")
        sid = lax.axis_index("s")
        gsems = (sem0, sem1)

        @pl.loop(0, RPT)
        def _(r):
            for l in range(DH // 16):
                buf[r, pl.ds(l * 16, 16)] = jnp.zeros((16,), jnp.float32)

        pltpu.sync_copy(buf, acc.at[pl.ds(sid * RPT, RPT)])
        plsc.subcore_barrier()

        def fetch_and_fire(chunk, b):
            base = sid * EPS + chunk * KE
            pltpu.sync_copy(src_hbm.at[pl.ds(base, KE)], sidx.at[b])
            pltpu.sync_copy(dst_hbm.at[pl.ds(base, KE)], didx.at[b])
            pltpu.async_copy(h_hbm.at[cid].at[sidx.at[b]], rows.at[b], gsems[b])

        fetch_and_fire(0, 0)
        fetch_and_fire(1, 1)

        @pl.loop(0, nchunk, step=2)
        def _(it):
            for b in range(2):
                pltpu.make_async_copy(
                    h_hbm.at[cid].at[sidx.at[b]], rows.at[b], gsems[b]
                ).wait()
                pltpu.sync_copy(rows.at[b], acc.at[didx.at[b]], add=True)

                @pl.when(it + b + 2 < nchunk)
                def _():
                    fetch_and_fire(it + b + 2, b)

        plsc.subcore_barrier()
        pltpu.sync_copy(acc.at[pl.ds(sid * RPT, RPT)], buf)
        pltpu.sync_copy(buf, out_hbm.at[cid, pl.ds(sid * RPT, RPT)])

    return k(h2, src, dst)


def _finalize_gf(partials):
    BM = 512

    def body(p_ref, o_ref):
        x0 = p_ref[0]
        x1 = p_ref[1]
        li = lax.broadcasted_iota(jnp.int32, (BM, DH), 1)
        deg = jnp.sum(jnp.where(li == 100 - DH, x1, 0.0), axis=1, keepdims=True)
        r = 1.0 / jnp.maximum(deg, 1.0)
        o_ref[...] = jnp.concatenate(
            [jnp.maximum(x0 * r, 0.0), jnp.maximum(x1 * r, 0.0)], axis=1
        )

    return pl.pallas_call(
        body,
        grid=(NPAD // BM,),
        in_specs=[pl.BlockSpec((NC, BM, DH), lambda i: (0, i, 0))],
        out_specs=pl.BlockSpec((BM, DP), lambda i: (i, 0)),
        out_shape=jax.ShapeDtypeStruct((NPAD, DP), jnp.float32),
    )(partials)


def _pair_diff(gf, pa, pb):
    mesh = plsc.VectorSubcoreMesh(core_axis_name="c", subcore_axis_name="s")
    nchunk = PPW // KP

    @functools.partial(
        pl.kernel,
        out_type=jax.ShapeDtypeStruct((PPAD, DP), jnp.float32),
        mesh=mesh,
        scratch_types=[
            pltpu.VMEM((2, KP), jnp.int32),
            pltpu.VMEM((2, KP), jnp.int32),
            pltpu.VMEM((2, KP, DP), jnp.float32),
            pltpu.VMEM((2, KP, DP), jnp.float32),
            pltpu.SemaphoreType.DMA,
            pltpu.SemaphoreType.DMA,
            pltpu.SemaphoreType.DMA,
            pltpu.SemaphoreType.DMA,
        ],
    )
    def k(gf_hbm, pa_hbm, pb_hbm, out_hbm, ia, ib, ra, rb,
          sa0, sa1, sb0, sb1):
        cid = lax.axis_index("c")
        sid = lax.axis_index("s")
        w = cid * NS + sid
        sA = (sa0, sa1)
        sB = (sb0, sb1)

        def fetch_and_fire(chunk, b):
            base = w * PPW + chunk * KP
            pltpu.sync_copy(pa_hbm.at[pl.ds(base, KP)], ia.at[b])
            pltpu.sync_copy(pb_hbm.at[pl.ds(base, KP)], ib.at[b])
            pltpu.async_copy(gf_hbm.at[ia.at[b]], ra.at[b], sA[b])
            pltpu.async_copy(gf_hbm.at[ib.at[b]], rb.at[b], sB[b])

        fetch_and_fire(0, 0)
        fetch_and_fire(1, 1)

        @pl.loop(0, nchunk, step=2)
        def _(it):
            for b in range(2):
                pltpu.make_async_copy(
                    gf_hbm.at[ia.at[b]], ra.at[b], sA[b]
                ).wait()
                pltpu.make_async_copy(
                    gf_hbm.at[ib.at[b]], rb.at[b], sB[b]
                ).wait()

                @pl.loop(0, KP, step=4)
                def _(r):
                    for dr in range(4):
                        for l in range(DP // 16):
                            sl = pl.ds(l * 16, 16)
                            ra[b, r + dr, sl] = jnp.abs(
                                ra[b, r + dr, sl] - rb[b, r + dr, sl]
                            )

                base = w * PPW + (it + b) * KP
                pltpu.sync_copy(ra.at[b], out_hbm.at[pl.ds(base, KP)])

                @pl.when(it + b + 2 < nchunk)
                def _():
                    fetch_and_fire(it + b + 2, b)

    return k(gf, pa, pb)


def _head_loss(dmat, W1p, b1p, w2p, b2, tvec):
    BM = 1024
    G = PPAD // BM
    SCALE = 1.0 / P

    def body(d_ref, w1_ref, b1_ref, w2_ref, b2_ref, t_ref, o_ref):
        i = pl.program_id(0)
        d = d_ref[...]
        hdn = jnp.maximum(
            jnp.dot(d, w1_ref[...], preferred_element_type=jnp.float32)
            + b1_ref[...],
            0.0,
        )
        lg = jnp.sum(hdn * w2_ref[...], axis=1, keepdims=True) + b2_ref[...]
        t = t_ref[...]
        row = i * BM + lax.broadcasted_iota(jnp.int32, (BM, 1), 0)
        wgt = jnp.where(row < P, SCALE, 0.0)
        bce = jnp.maximum(lg, 0.0) - lg * t + jnp.log1p(jnp.exp(-jnp.abs(lg)))
        part = jnp.reshape(jnp.sum(bce * wgt), (1, 1))

        @pl.when(i == 0)
        def _():
            o_ref[...] = part

        @pl.when(i > 0)
        def _():
            o_ref[...] += part

    return pl.pallas_call(
        body,
        grid=(G,),
        in_specs=[
            pl.BlockSpec((BM, DP), lambda i: (i, 0)),
            pl.BlockSpec((DP, DP), lambda i: (0, 0)),
            pl.BlockSpec((1, DP), lambda i: (0, 0)),
            pl.BlockSpec((1, DP), lambda i: (0, 0)),
            pl.BlockSpec((1, 1), lambda i: (0, 0)),
            pl.BlockSpec((BM, 1), lambda i: (i, 0)),
        ],
        out_specs=pl.BlockSpec((1, 1), lambda i: (0, 0)),
        out_shape=jax.ShapeDtypeStruct((1, 1), jnp.float32),
    )(dmat, W1p, b1p, w2p, b2, tvec)


def kernel(vertex_features, edge_index, pairs_cells, pairs_cols, pairs_rows,
           targets_cells, targets_cols, targets_rows,
           W_gcnn, b_gcnn, W_h1, b_h1, W_h2, b_h2):
    f32 = jnp.float32
    src = edge_index[0]
    dst = edge_index[1]

    Wp = jnp.pad(W_gcnn, ((0, 0), (0, DP - 100)))
    bp = jnp.concatenate(
        [b_gcnn, jnp.ones((1,), f32), jnp.zeros((DP - 101,), f32)]
    ).reshape(1, DP)
    W1p = jnp.pad(W_h1, ((0, DP - 100), (0, DP - 50)))
    b1p = jnp.pad(b_h1, (0, DP - 50)).reshape(1, DP)
    w2p = jnp.pad(W_h2[:, 0], (0, DP - 50)).reshape(1, DP)
    b2 = b_h2.reshape(1, 1)

    def padset(x):
        return jnp.pad(x, (0, PPAD - P))

    h2 = _linear(vertex_features, Wp, bp)
    partials = _edge_agg(h2, src, dst)
    gf = _finalize_gf(partials)

    total = None
    for pairs, targets in (
        (pairs_cells, targets_cells),
        (pairs_cols, targets_cols),
        (pairs_rows, targets_rows),
    ):
        pa = padset(pairs[:, 0])
        pb = padset(pairs[:, 1])
        tvec = padset(targets.astype(f32)).reshape(PPAD, 1)
        dmat = _pair_diff(gf, pa, pb)
        loss = _head_loss(dmat, W1p, b1p, w2p, b2, tvec)[0, 0]
        total = loss if total is None else total + loss
    return total
